# Initial kernel scaffold; baseline (speedup 1.0000x reference)
import functools

import jax
import jax.numpy as jnp
from jax import lax
from jax.experimental import pallas as pl
from jax.experimental.pallas import tpu as pltpu
from jax.experimental.pallas import tpu_sc as plsc

N = 10000
E = 320000
D = 128


_TC_GRID = 25
_NROWS = N // _TC_GRID
_EROWS = E // _TC_GRID


def _tc_fwd_body(feat_ref, w_ref, b_ref, att_ref, hw_ref, att16_ref):
    hw_ref[...] = (
        jnp.dot(feat_ref[...], w_ref[...], preferred_element_type=jnp.float32)
        + b_ref[...]
    )
    att16_ref[...] = jnp.broadcast_to(att_ref[...], (_EROWS, 16))


def _tc_fwd(feat, w, b2d, att):
    return pl.pallas_call(
        _tc_fwd_body,
        grid=(_TC_GRID,),
        in_specs=[
            pl.BlockSpec((_NROWS, D), lambda i: (i, 0)),
            pl.BlockSpec((D, D), lambda i: (0, 0)),
            pl.BlockSpec((1, D), lambda i: (0, 0)),
            pl.BlockSpec((_EROWS, 1), lambda i: (i, 0)),
        ],
        out_specs=[
            pl.BlockSpec((_NROWS, D), lambda i: (i, 0)),
            pl.BlockSpec((_EROWS, 16), lambda i: (i, 0)),
        ],
        out_shape=[
            jax.ShapeDtypeStruct((N, D), jnp.float32),
            jax.ShapeDtypeStruct((E, 16), jnp.float32),
        ],
    )(feat, w, b2d, att)


_NTILES = 32
_EDGES_PER_TILE = E // _NTILES
_B = 80
_NBATCH = _EDGES_PER_TILE // _B
_ROWS_PER_TILE = N // 16
_ZCHUNK = 125


def _sc_scatter(hw, src, dst, att16):
    mesh = plsc.VectorSubcoreMesh(core_axis_name="c", subcore_axis_name="s")

    @functools.partial(
        pl.kernel,
        out_type=jax.ShapeDtypeStruct((2, N, D), jnp.float32),
        mesh=mesh,
        scratch_types=[
            pltpu.VMEM((_B,), jnp.int32),
            pltpu.VMEM((_B,), jnp.int32),
            pltpu.VMEM((_B, 16), jnp.float32),
            pltpu.VMEM((_B, D), jnp.float32),
            pltpu.VMEM((_ZCHUNK, D), jnp.float32),
            pltpu.VMEM_SHARED((N, D), jnp.float32),
        ],
    )
    def body(hw_hbm, src_hbm, dst_hbm, att_hbm, out_hbm,
             srcv, dstv, attv, rowsv, zbuf, acc):
        cid = lax.axis_index("c")
        sid = lax.axis_index("s")
        wid = cid * 16 + sid

        @pl.loop(0, _ZCHUNK)
        def _(r):
            for c in range(D // 16):
                zbuf[r, pl.ds(c * 16, 16)] = jnp.zeros((16,), jnp.float32)

        for k in range(_ROWS_PER_TILE // _ZCHUNK):
            r0 = sid * _ROWS_PER_TILE + k * _ZCHUNK
            pltpu.sync_copy(zbuf, acc.at[pl.ds(r0, _ZCHUNK)])
        plsc.subcore_barrier()

        base = wid * _EDGES_PER_TILE

        @pl.loop(0, _NBATCH)
        def _(j):
            off = base + j * _B
            pltpu.sync_copy(src_hbm.at[pl.ds(off, _B)], srcv)
            pltpu.sync_copy(dst_hbm.at[pl.ds(off, _B)], dstv)
            pltpu.sync_copy(att_hbm.at[pl.ds(off, _B)], attv)
            pltpu.sync_copy(hw_hbm.at[srcv], rowsv)

            @pl.loop(0, _B)
            def _(e):
                a = attv[e]
                for c in range(D // 16):
                    rowsv[e, pl.ds(c * 16, 16)] = rowsv[e, pl.ds(c * 16, 16)] * a

            pltpu.sync_copy(rowsv, acc.at[dstv], add=True)

        plsc.subcore_barrier()

        for k in range(_ROWS_PER_TILE // _ZCHUNK):
            r0 = sid * _ROWS_PER_TILE + k * _ZCHUNK
            pltpu.sync_copy(acc.at[pl.ds(r0, _ZCHUNK)], zbuf)
            pltpu.sync_copy(zbuf, out_hbm.at[cid, pl.ds(r0, _ZCHUNK)])

    return body(hw, src, dst, att16)


def _tc_add_body(p_ref, o_ref):
    o_ref[...] = p_ref[0] + p_ref[1]


def _tc_add(partials):
    return pl.pallas_call(
        _tc_add_body,
        grid=(_TC_GRID,),
        in_specs=[pl.BlockSpec((2, _NROWS, D), lambda i: (0, i, 0))],
        out_specs=pl.BlockSpec((_NROWS, D), lambda i: (i, 0)),
        out_shape=jax.ShapeDtypeStruct((N, D), jnp.float32),
    )(partials)


def kernel(feat, edge_index, attention, W, b):
    src = edge_index[0].astype(jnp.int32)
    dst = edge_index[1].astype(jnp.int32)
    hw, att16 = _tc_fwd(feat, W, b.reshape(1, D), attention.reshape(E, 1))
    partials = _sc_scatter(hw, src, dst, att16)
    return _tc_add(partials)


# speedup vs baseline: 2.8818x; 2.8818x over previous
import functools

import jax
import jax.numpy as jnp
from jax import lax
from jax.experimental import pallas as pl
from jax.experimental.pallas import tpu as pltpu
from jax.experimental.pallas import tpu_sc as plsc

N = 10000
E = 320000
D = 128


_TC_GRID = 25
_NROWS = N // _TC_GRID
_EROWS = E // _TC_GRID


def _tc_fwd_body(feat_ref, w_ref, b_ref, att_ref, hw_ref, att16_ref):
    hw_ref[...] = (
        jnp.dot(feat_ref[...], w_ref[...], preferred_element_type=jnp.float32)
        + b_ref[...]
    )
    att16_ref[...] = jnp.broadcast_to(att_ref[...], (_EROWS, 16))


def _tc_fwd(feat, w, b2d, att):
    return pl.pallas_call(
        _tc_fwd_body,
        grid=(_TC_GRID,),
        in_specs=[
            pl.BlockSpec((_NROWS, D), lambda i: (i, 0)),
            pl.BlockSpec((D, D), lambda i: (0, 0)),
            pl.BlockSpec((1, D), lambda i: (0, 0)),
            pl.BlockSpec((_EROWS, 1), lambda i: (i, 0)),
        ],
        out_specs=[
            pl.BlockSpec((_NROWS, D), lambda i: (i, 0)),
            pl.BlockSpec((_EROWS, 16), lambda i: (i, 0)),
        ],
        out_shape=[
            jax.ShapeDtypeStruct((N, D), jnp.float32),
            jax.ShapeDtypeStruct((E, 16), jnp.float32),
        ],
    )(feat, w, b2d, att)


_NTILES = 32
_EDGES_PER_TILE = E // _NTILES
_B = 80
_NBATCH = _EDGES_PER_TILE // _B
_ACC_ROWS = 10240
_STRIPE = _ACC_ROWS // 16
_ZROWS = 128
_WCHUNK = 80


def _sc_scatter(hw, src, dst, att16):
    mesh = plsc.VectorSubcoreMesh(core_axis_name="c", subcore_axis_name="s")

    @functools.partial(
        pl.kernel,
        out_type=jax.ShapeDtypeStruct((2, N, D), jnp.float32),
        mesh=mesh,
        scratch_types=[
            pltpu.VMEM((_B,), jnp.int32),
            pltpu.VMEM((_B,), jnp.int32),
            pltpu.VMEM((_B, 16), jnp.float32),
            pltpu.VMEM((_B, D), jnp.float32),
            pltpu.VMEM((_ZROWS, D), jnp.float32),
            pltpu.VMEM_SHARED((_ACC_ROWS, D), jnp.float32),
        ],
    )
    def body(hw_hbm, src_hbm, dst_hbm, att_hbm, out_hbm,
             srcv, dstv, attv, rowsv, zbuf, acc):
        cid = lax.axis_index("c")
        sid = lax.axis_index("s")
        wid = cid * 16 + sid

        @pl.loop(0, _ZROWS)
        def _(r):
            for c in range(D // 16):
                zbuf[r, pl.ds(c * 16, 16)] = jnp.zeros((16,), jnp.float32)

        for k in range(_STRIPE // _ZROWS):
            r0 = sid * _STRIPE + k * _ZROWS
            pltpu.sync_copy(zbuf, acc.at[pl.ds(r0, _ZROWS)])
        plsc.subcore_barrier()

        base = wid * _EDGES_PER_TILE

        @pl.loop(0, _NBATCH)
        def _(j):
            off = base + j * _B
            pltpu.sync_copy(src_hbm.at[pl.ds(off, _B)], srcv)
            pltpu.sync_copy(dst_hbm.at[pl.ds(off, _B)], dstv)
            pltpu.sync_copy(att_hbm.at[pl.ds(off, _B)], attv)
            pltpu.sync_copy(hw_hbm.at[srcv], rowsv)

            @pl.loop(0, _B)
            def _(e):
                a = attv[e]
                for c in range(D // 16):
                    rowsv[e, pl.ds(c * 16, 16)] = rowsv[e, pl.ds(c * 16, 16)] * a

            pltpu.sync_copy(rowsv, acc.at[dstv], add=True)

        plsc.subcore_barrier()

        for k in range(_STRIPE // _WCHUNK):
            r0 = sid * _STRIPE + k * _WCHUNK

            @pl.when(r0 < N)
            def _():
                pltpu.sync_copy(acc.at[pl.ds(r0, _WCHUNK)],
                                zbuf.at[pl.ds(0, _WCHUNK)])
                pltpu.sync_copy(zbuf.at[pl.ds(0, _WCHUNK)],
                                out_hbm.at[cid, pl.ds(r0, _WCHUNK)])

    return body(hw, src, dst, att16)


def _tc_add_body(p_ref, o_ref):
    o_ref[...] = p_ref[0] + p_ref[1]


def _tc_add(partials):
    return pl.pallas_call(
        _tc_add_body,
        grid=(_TC_GRID,),
        in_specs=[pl.BlockSpec((2, _NROWS, D), lambda i: (0, i, 0))],
        out_specs=pl.BlockSpec((_NROWS, D), lambda i: (i, 0)),
        out_shape=jax.ShapeDtypeStruct((N, D), jnp.float32),
    )(partials)


def kernel(feat, edge_index, attention, W, b):
    src = edge_index[0].astype(jnp.int32)
    dst = edge_index[1].astype(jnp.int32)
    hw, att16 = _tc_fwd(feat, W, b.reshape(1, D), attention.reshape(E, 1))
    partials = _sc_scatter(hw, src, dst, att16)
    return _tc_add(partials)


# speedup vs baseline: 4.0424x; 1.4027x over previous
import functools

import jax
import jax.numpy as jnp
from jax import lax
from jax.experimental import pallas as pl
from jax.experimental.pallas import tpu as pltpu
from jax.experimental.pallas import tpu_sc as plsc

N = 10000
E = 320000
D = 128


_TC_GRID = 25
_NROWS = N // _TC_GRID
_EROWS = E // _TC_GRID


def _tc_fwd_body(feat_ref, w_ref, b_ref, att_ref, hw_ref, att16_ref):
    hw_ref[...] = (
        jnp.dot(feat_ref[...], w_ref[...], preferred_element_type=jnp.float32)
        + b_ref[...]
    )
    att16_ref[...] = jnp.broadcast_to(att_ref[...], (_EROWS, 16))


def _tc_fwd(feat, w, b2d, att):
    return pl.pallas_call(
        _tc_fwd_body,
        grid=(_TC_GRID,),
        in_specs=[
            pl.BlockSpec((_NROWS, D), lambda i: (i, 0)),
            pl.BlockSpec((D, D), lambda i: (0, 0)),
            pl.BlockSpec((1, D), lambda i: (0, 0)),
            pl.BlockSpec((_EROWS, 1), lambda i: (i, 0)),
        ],
        out_specs=[
            pl.BlockSpec((_NROWS, D), lambda i: (i, 0)),
            pl.BlockSpec((_EROWS, 16), lambda i: (i, 0)),
        ],
        out_shape=[
            jax.ShapeDtypeStruct((N, D), jnp.float32),
            jax.ShapeDtypeStruct((E, 16), jnp.float32),
        ],
    )(feat, w, b2d, att)


_NTILES = 32
_EDGES_PER_TILE = E // _NTILES
_B = 80
_NBATCH = _EDGES_PER_TILE // _B
_ACC_ROWS = 10240
_STRIPE = _ACC_ROWS // 16
_ZROWS = 80
_WCHUNK = 80


def _sc_scatter(hw, src, dst, att16):
    mesh = plsc.VectorSubcoreMesh(core_axis_name="c", subcore_axis_name="s")

    @functools.partial(
        pl.kernel,
        out_type=jax.ShapeDtypeStruct((2, N, D), jnp.float32),
        mesh=mesh,
        scratch_types=[
            pltpu.VMEM((_B,), jnp.int32),
            pltpu.VMEM((_B,), jnp.int32),
            pltpu.VMEM((_B,), jnp.int32),
            pltpu.VMEM((_B,), jnp.int32),
            pltpu.VMEM((_B,), jnp.int32),
            pltpu.VMEM((_B,), jnp.int32),
            pltpu.VMEM((_B * 16,), jnp.float32),
            pltpu.VMEM((_B * 16,), jnp.float32),
            pltpu.VMEM((_B, D), jnp.float32),
            pltpu.VMEM((_B, D), jnp.float32),
            pltpu.VMEM((_ZROWS, D), jnp.float32),
            pltpu.VMEM_SHARED((_ACC_ROWS, D), jnp.float32),
            pltpu.SemaphoreType.DMA,
            pltpu.SemaphoreType.DMA,
            pltpu.SemaphoreType.DMA,
            pltpu.SemaphoreType.DMA,
            pltpu.SemaphoreType.DMA,
            pltpu.SemaphoreType.DMA,
        ],
    )
    def body(hw_hbm, src_hbm, dst_hbm, att_hbm, out_hbm,
             srcv0, srcv1, dstv0, dstv1, dsts0, dsts1, attv0, attv1,
             rows0, rows1, zbuf, acc,
             sem_i0, sem_i1, sem_g0, sem_g1, sem_s0, sem_s1):
        cid = lax.axis_index("c")
        sid = lax.axis_index("s")
        wid = cid * 16 + sid

        slots = (
            (srcv0, dstv0, dsts0, attv0, rows0, sem_i0, sem_g0, sem_s0),
            (srcv1, dstv1, dsts1, attv1, rows1, sem_i1, sem_g1, sem_s1),
        )

        @pl.loop(0, _ZROWS)
        def _(r):
            for c in range(D // 16):
                zbuf[r, pl.ds(c * 16, 16)] = jnp.zeros((16,), jnp.float32)

        for k in range(_STRIPE // _ZROWS):
            r0 = sid * _STRIPE + k * _ZROWS
            pltpu.sync_copy(zbuf, acc.at[pl.ds(r0, _ZROWS)])
        plsc.subcore_barrier()

        base = wid * _EDGES_PER_TILE

        def idx_start(j, s):
            srcv, dstv, _, attv, _, sem_i, _, _ = slots[s]
            off = base + j * _B
            pltpu.async_copy(src_hbm.at[pl.ds(off, _B)], srcv, sem_i)
            pltpu.async_copy(dst_hbm.at[pl.ds(off, _B)], dstv, sem_i)
            pltpu.async_copy(att_hbm.at[pl.ds(off * 16, _B * 16)], attv, sem_i)

        def idx_wait(s):
            srcv, dstv, _, attv, _, sem_i, _, _ = slots[s]
            pltpu.make_async_copy(src_hbm.at[pl.ds(0, _B)], srcv, sem_i).wait()
            pltpu.make_async_copy(dst_hbm.at[pl.ds(0, _B)], dstv, sem_i).wait()
            pltpu.make_async_copy(att_hbm.at[pl.ds(0, _B * 16)], attv, sem_i).wait()

        def g_start(s):
            srcv, _, _, _, rows, _, sem_g, _ = slots[s]
            pltpu.async_copy(hw_hbm.at[srcv], rows, sem_g)

        def g_wait(s):
            srcv, _, _, _, rows, _, sem_g, _ = slots[s]
            pltpu.make_async_copy(hw_hbm.at[srcv], rows, sem_g).wait()

        def compute_and_scatter(s):
            _, dstv, dsts, attv, rows, _, _, sem_s = slots[s]

            @pl.loop(0, _B)
            def _(e):
                a = attv[pl.ds(e * 16, 16)]
                for c in range(D // 16):
                    rows[e, pl.ds(c * 16, 16)] = rows[e, pl.ds(c * 16, 16)] * a

            for k in range(_B // 16):
                dsts[pl.ds(k * 16, 16)] = dstv[pl.ds(k * 16, 16)]
            pltpu.async_copy(rows, acc.at[dsts], sem_s, add=True)

        def s_wait(s):
            _, _, dsts, _, rows, _, _, sem_s = slots[s]
            pltpu.make_async_copy(rows, acc.at[dsts], sem_s).wait()

        idx_start(0, 0)
        idx_start(1, 1)
        idx_wait(0)
        g_start(0)
        idx_wait(1)
        g_start(1)
        g_wait(0)
        compute_and_scatter(0)
        idx_start(2, 0)
        g_wait(1)
        compute_and_scatter(1)
        idx_start(3, 1)

        @pl.loop(2, _NBATCH - 1, step=2)
        def _(j):
            idx_wait(0)
            s_wait(0)
            g_start(0)
            idx_wait(1)
            s_wait(1)
            g_start(1)
            g_wait(0)
            compute_and_scatter(0)
            idx_start(j + 2, 0)
            g_wait(1)
            compute_and_scatter(1)

            @pl.when(j < _NBATCH - 4)
            def _():
                idx_start(j + 3, 1)

        idx_wait(0)
        s_wait(0)
        g_start(0)
        g_wait(0)
        compute_and_scatter(0)
        s_wait(0)
        s_wait(1)

        plsc.subcore_barrier()

        for k in range(_STRIPE // _WCHUNK):
            r0 = sid * _STRIPE + k * _WCHUNK

            @pl.when(r0 < N)
            def _():
                pltpu.sync_copy(acc.at[pl.ds(r0, _WCHUNK)],
                                zbuf.at[pl.ds(0, _WCHUNK)])
                pltpu.sync_copy(zbuf.at[pl.ds(0, _WCHUNK)],
                                out_hbm.at[cid, pl.ds(r0, _WCHUNK)])

    return body(hw, src, dst, att16)


def _tc_add_body(p_ref, o_ref):
    o_ref[...] = p_ref[0] + p_ref[1]


def _tc_add(partials):
    return pl.pallas_call(
        _tc_add_body,
        grid=(_TC_GRID,),
        in_specs=[pl.BlockSpec((2, _NROWS, D), lambda i: (0, i, 0))],
        out_specs=pl.BlockSpec((_NROWS, D), lambda i: (i, 0)),
        out_shape=jax.ShapeDtypeStruct((N, D), jnp.float32),
    )(partials)


def kernel(feat, edge_index, attention, W, b):
    src = edge_index[0].astype(jnp.int32)
    dst = edge_index[1].astype(jnp.int32)
    hw, att16 = _tc_fwd(feat, W, b.reshape(1, D), attention.reshape(E, 1))
    partials = _sc_scatter(hw, src, dst, att16.reshape(E * 16))
    return _tc_add(partials)


# speedup vs baseline: 7.7431x; 1.9155x over previous
import dataclasses
import functools

import jax
import jax.numpy as jnp
from jax import lax
from jax.experimental import pallas as pl
from jax.experimental.pallas import tpu as pltpu
from jax.experimental.pallas import tpu_sc as plsc

N = 10000
E = 320000
D = 128


_TC_GRID = 25
_NROWS = N // _TC_GRID
_EROWS = E // _TC_GRID


def _tc_fwd_body(feat_ref, w_ref, b_ref, hw_ref):
    hw_ref[...] = (
        jnp.dot(feat_ref[...], w_ref[...], preferred_element_type=jnp.float32)
        + b_ref[...]
    )


def _tc_fwd(feat, w, b2d):
    return pl.pallas_call(
        _tc_fwd_body,
        grid=(_TC_GRID,),
        in_specs=[
            pl.BlockSpec((_NROWS, D), lambda i: (i, 0)),
            pl.BlockSpec((D, D), lambda i: (0, 0)),
            pl.BlockSpec((1, D), lambda i: (0, 0)),
        ],
        out_specs=pl.BlockSpec((_NROWS, D), lambda i: (i, 0)),
        out_shape=jax.ShapeDtypeStruct((N, D), jnp.float32),
    )(feat, w, b2d)


_NTILES = 32
_EDGES_PER_TILE = E // _NTILES
_B = 80
_NBATCH = _EDGES_PER_TILE // _B
_ACC_ROWS = 10240
_STRIPE = _ACC_ROWS // 16
_ZROWS = 80
_WCHUNK = 80


def _sc_scatter(hw, src, dst, att16):
    mesh = plsc.VectorSubcoreMesh(core_axis_name="c", subcore_axis_name="s")
    cp = pltpu.CompilerParams()
    if "needs_layout_passes" in pltpu.CompilerParams.__dataclass_fields__:
        cp = dataclasses.replace(cp, needs_layout_passes=False)

    @functools.partial(
        pl.kernel,
        out_type=jax.ShapeDtypeStruct((2, N, D), jnp.float32),
        mesh=mesh,
        compiler_params=cp,
        scratch_types=[
            pltpu.VMEM((_B,), jnp.int32),
            pltpu.VMEM((_B,), jnp.int32),
            pltpu.VMEM((_B,), jnp.int32),
            pltpu.VMEM((_B,), jnp.int32),
            pltpu.VMEM((_B,), jnp.int32),
            pltpu.VMEM((_B,), jnp.int32),
            pltpu.VMEM((_B,), jnp.float32),
            pltpu.VMEM((_B,), jnp.float32),
            pltpu.VMEM((_B, D), jnp.float32),
            pltpu.VMEM((_B, D), jnp.float32),
            pltpu.VMEM((_ZROWS, D), jnp.float32),
            pltpu.VMEM_SHARED((_ACC_ROWS, D), jnp.float32),
            pltpu.SemaphoreType.DMA,
            pltpu.SemaphoreType.DMA,
            pltpu.SemaphoreType.DMA,
            pltpu.SemaphoreType.DMA,
            pltpu.SemaphoreType.DMA,
            pltpu.SemaphoreType.DMA,
        ],
    )
    def body(hw_hbm, src_hbm, dst_hbm, att_hbm, out_hbm,
             srcv0, srcv1, dstv0, dstv1, dsts0, dsts1, attv0, attv1,
             rows0, rows1, zbuf, acc,
             sem_i0, sem_i1, sem_g0, sem_g1, sem_s0, sem_s1):
        cid = lax.axis_index("c")
        sid = lax.axis_index("s")
        wid = cid * 16 + sid

        slots = (
            (srcv0, dstv0, dsts0, attv0, rows0, sem_i0, sem_g0, sem_s0),
            (srcv1, dstv1, dsts1, attv1, rows1, sem_i1, sem_g1, sem_s1),
        )

        @pl.loop(0, _ZROWS)
        def _(r):
            for c in range(D // 16):
                zbuf[r, pl.ds(c * 16, 16)] = jnp.zeros((16,), jnp.float32)

        for k in range(_STRIPE // _ZROWS):
            r0 = sid * _STRIPE + k * _ZROWS
            pltpu.sync_copy(zbuf, acc.at[pl.ds(r0, _ZROWS)])
        plsc.subcore_barrier()

        base = wid * _EDGES_PER_TILE

        def idx_start(j, s):
            srcv, dstv, _, attv, _, sem_i, _, _ = slots[s]
            off = base + j * _B
            pltpu.async_copy(src_hbm.at[pl.ds(off, _B)], srcv, sem_i)
            pltpu.async_copy(dst_hbm.at[pl.ds(off, _B)], dstv, sem_i)
            pltpu.async_copy(att_hbm.at[pl.ds(off, _B)], attv, sem_i)

        def idx_wait(s):
            srcv, dstv, _, attv, _, sem_i, _, _ = slots[s]
            pltpu.make_async_copy(src_hbm.at[pl.ds(0, _B)], srcv, sem_i).wait()
            pltpu.make_async_copy(dst_hbm.at[pl.ds(0, _B)], dstv, sem_i).wait()
            pltpu.make_async_copy(att_hbm.at[pl.ds(0, _B)], attv, sem_i).wait()

        def g_start(s):
            srcv, _, _, _, rows, _, sem_g, _ = slots[s]
            pltpu.async_copy(hw_hbm.at[srcv], rows, sem_g)

        def g_wait(s):
            srcv, _, _, _, rows, _, sem_g, _ = slots[s]
            pltpu.make_async_copy(hw_hbm.at[srcv], rows, sem_g).wait()

        def compute_and_scatter(s):
            _, dstv, dsts, attv, rows, _, _, sem_s = slots[s]

            @pl.loop(0, _B)
            def _(e):
                a = plsc.load_gather(attv, [jnp.full((16,), e, jnp.int32)])
                for c in range(D // 16):
                    rows[e, pl.ds(c * 16, 16)] = rows[e, pl.ds(c * 16, 16)] * a

            for k in range(_B // 16):
                dsts[pl.ds(k * 16, 16)] = dstv[pl.ds(k * 16, 16)]
            pltpu.async_copy(rows, acc.at[dsts], sem_s, add=True)

        def s_wait(s):
            _, _, dsts, _, rows, _, _, sem_s = slots[s]
            pltpu.make_async_copy(rows, acc.at[dsts], sem_s).wait()

        idx_start(0, 0)
        idx_start(1, 1)
        idx_wait(0)
        g_start(0)
        idx_wait(1)
        g_start(1)
        g_wait(0)
        compute_and_scatter(0)
        idx_start(2, 0)
        g_wait(1)
        compute_and_scatter(1)
        idx_start(3, 1)

        @pl.loop(2, _NBATCH - 1, step=2)
        def _(j):
            idx_wait(0)
            s_wait(0)
            g_start(0)
            idx_wait(1)
            s_wait(1)
            g_start(1)
            g_wait(0)
            compute_and_scatter(0)
            idx_start(j + 2, 0)
            g_wait(1)
            compute_and_scatter(1)

            @pl.when(j < _NBATCH - 4)
            def _():
                idx_start(j + 3, 1)

        idx_wait(0)
        s_wait(0)
        g_start(0)
        g_wait(0)
        compute_and_scatter(0)
        s_wait(0)
        s_wait(1)

        plsc.subcore_barrier()

        for k in range(_STRIPE // _WCHUNK):
            r0 = sid * _STRIPE + k * _WCHUNK

            @pl.when(r0 < N)
            def _():
                pltpu.sync_copy(acc.at[pl.ds(r0, _WCHUNK)],
                                zbuf.at[pl.ds(0, _WCHUNK)])
                pltpu.sync_copy(zbuf.at[pl.ds(0, _WCHUNK)],
                                out_hbm.at[cid, pl.ds(r0, _WCHUNK)])

    return body(hw, src, dst, att16)


def _tc_add_body(p_ref, o_ref):
    o_ref[...] = p_ref[0] + p_ref[1]


def _tc_add(partials):
    return pl.pallas_call(
        _tc_add_body,
        grid=(_TC_GRID,),
        in_specs=[pl.BlockSpec((2, _NROWS, D), lambda i: (0, i, 0))],
        out_specs=pl.BlockSpec((_NROWS, D), lambda i: (i, 0)),
        out_shape=jax.ShapeDtypeStruct((N, D), jnp.float32),
    )(partials)


def kernel(feat, edge_index, attention, W, b):
    src = edge_index[0].astype(jnp.int32)
    dst = edge_index[1].astype(jnp.int32)
    hw = _tc_fwd(feat, W, b.reshape(1, D))
    partials = _sc_scatter(hw, src, dst, attention.reshape(E))
    return _tc_add(partials)


# speedup vs baseline: 9.2327x; 1.1924x over previous
import dataclasses
import functools

import jax
import jax.numpy as jnp
from jax import lax
from jax.experimental import pallas as pl
from jax.experimental.pallas import tpu as pltpu
from jax.experimental.pallas import tpu_sc as plsc

N = 10000
E = 320000
D = 128


_TC_GRID = 25
_NROWS = N // _TC_GRID
_EROWS = E // _TC_GRID


def _tc_fwd_body(feat_ref, w_ref, b_ref, hw_ref):
    hw_ref[...] = (
        jnp.dot(feat_ref[...], w_ref[...], preferred_element_type=jnp.float32)
        + b_ref[...]
    )


def _tc_fwd(feat, w, b2d):
    return pl.pallas_call(
        _tc_fwd_body,
        grid=(_TC_GRID,),
        in_specs=[
            pl.BlockSpec((_NROWS, D), lambda i: (i, 0)),
            pl.BlockSpec((D, D), lambda i: (0, 0)),
            pl.BlockSpec((1, D), lambda i: (0, 0)),
        ],
        out_specs=pl.BlockSpec((_NROWS, D), lambda i: (i, 0)),
        out_shape=jax.ShapeDtypeStruct((N, D), jnp.float32),
    )(feat, w, b2d)


_GATHER_DNUMS = lax.GatherDimensionNumbers(
    offset_dims=(), collapsed_slice_dims=(0,), start_index_map=(0,))

_NTILES = 32
_EDGES_PER_TILE = E // _NTILES
_B = 80
_NBATCH = _EDGES_PER_TILE // _B
_ACC_ROWS = 10240
_STRIPE = _ACC_ROWS // 16
_ZROWS = 80
_WCHUNK = 80


def _sc_scatter(hw, ei_flat, att_flat):
    mesh = plsc.VectorSubcoreMesh(core_axis_name="c", subcore_axis_name="s")
    cp = pltpu.CompilerParams()
    if "needs_layout_passes" in pltpu.CompilerParams.__dataclass_fields__:
        cp = dataclasses.replace(cp, needs_layout_passes=False)

    @functools.partial(
        pl.kernel,
        out_type=jax.ShapeDtypeStruct((2, N, D), jnp.float32),
        mesh=mesh,
        compiler_params=cp,
        scratch_types=[
            pltpu.VMEM((_B,), jnp.int32),
            pltpu.VMEM((_B,), jnp.int32),
            pltpu.VMEM((_B,), jnp.int32),
            pltpu.VMEM((_B,), jnp.int32),
            pltpu.VMEM((_B,), jnp.int32),
            pltpu.VMEM((_B,), jnp.int32),
            pltpu.VMEM((_B,), jnp.float32),
            pltpu.VMEM((_B,), jnp.float32),
            pltpu.VMEM((_B, D), jnp.float32),
            pltpu.VMEM((_B, D), jnp.float32),
            pltpu.VMEM((_ZROWS, D), jnp.float32),
            pltpu.VMEM_SHARED((_ACC_ROWS, D), jnp.float32),
            pltpu.SemaphoreType.DMA,
            pltpu.SemaphoreType.DMA,
            pltpu.SemaphoreType.DMA,
            pltpu.SemaphoreType.DMA,
            pltpu.SemaphoreType.DMA,
            pltpu.SemaphoreType.DMA,
        ],
    )
    def body(hw_hbm, ei_hbm, att_hbm, out_hbm,
             srcv0, srcv1, dstv0, dstv1, dsts0, dsts1, attv0, attv1,
             rows0, rows1, zbuf, acc,
             sem_i0, sem_i1, sem_g0, sem_g1, sem_s0, sem_s1):
        cid = lax.axis_index("c")
        sid = lax.axis_index("s")
        wid = cid * 16 + sid

        slots = (
            (srcv0, dstv0, dsts0, attv0, rows0, sem_i0, sem_g0, sem_s0),
            (srcv1, dstv1, dsts1, attv1, rows1, sem_i1, sem_g1, sem_s1),
        )

        @pl.loop(0, _ZROWS)
        def _(r):
            for c in range(D // 16):
                zbuf[r, pl.ds(c * 16, 16)] = jnp.zeros((16,), jnp.float32)

        for k in range(_STRIPE // _ZROWS):
            r0 = sid * _STRIPE + k * _ZROWS
            pltpu.sync_copy(zbuf, acc.at[pl.ds(r0, _ZROWS)])
        plsc.subcore_barrier()

        base = wid * _EDGES_PER_TILE

        def idx_start(j, s):
            srcv, dstv, _, attv, _, sem_i, _, _ = slots[s]
            off = base + j * _B
            pltpu.async_copy(ei_hbm.at[pl.ds(off, _B)], srcv, sem_i)
            pltpu.async_copy(ei_hbm.at[pl.ds(E + off, _B)], dstv, sem_i)
            pltpu.async_copy(att_hbm.at[pl.ds(off, _B)], attv, sem_i)

        def idx_wait(s):
            srcv, dstv, _, attv, _, sem_i, _, _ = slots[s]
            pltpu.make_async_copy(ei_hbm.at[pl.ds(0, _B)], srcv, sem_i).wait()
            pltpu.make_async_copy(ei_hbm.at[pl.ds(0, _B)], dstv, sem_i).wait()
            pltpu.make_async_copy(att_hbm.at[pl.ds(0, _B)], attv, sem_i).wait()

        def g_start(s):
            srcv, _, _, _, rows, _, sem_g, _ = slots[s]
            pltpu.async_copy(hw_hbm.at[srcv], rows, sem_g)

        def g_wait(s):
            srcv, _, _, _, rows, _, sem_g, _ = slots[s]
            pltpu.make_async_copy(hw_hbm.at[srcv], rows, sem_g).wait()

        def compute_and_scatter(s):
            _, dstv, dsts, attv, rows, _, _, sem_s = slots[s]

            @pl.loop(0, _B // 16)
            def _(g):
                a16 = attv[pl.ds(g * 16, 16)]
                for e2 in range(16):
                    a = lax.gather(
                        a16, jnp.full((16, 1), e2, jnp.int32),
                        _GATHER_DNUMS, slice_sizes=(1,),
                        mode=lax.GatherScatterMode.PROMISE_IN_BOUNDS)
                    e = g * 16 + e2
                    for c in range(D // 16):
                        rows[e, pl.ds(c * 16, 16)] = (
                            rows[e, pl.ds(c * 16, 16)] * a)

            for k in range(_B // 16):
                dsts[pl.ds(k * 16, 16)] = dstv[pl.ds(k * 16, 16)]
            pltpu.async_copy(rows, acc.at[dsts], sem_s, add=True)

        def s_wait(s):
            _, _, dsts, _, rows, _, _, sem_s = slots[s]
            pltpu.make_async_copy(rows, acc.at[dsts], sem_s).wait()

        idx_start(0, 0)
        idx_start(1, 1)
        idx_wait(0)
        g_start(0)
        idx_wait(1)
        g_start(1)
        g_wait(0)
        compute_and_scatter(0)
        idx_start(2, 0)
        g_wait(1)
        compute_and_scatter(1)
        idx_start(3, 1)

        @pl.loop(2, _NBATCH - 1, step=2)
        def _(j):
            idx_wait(0)
            s_wait(0)
            g_start(0)
            idx_wait(1)
            s_wait(1)
            g_start(1)
            g_wait(0)
            compute_and_scatter(0)
            idx_start(j + 2, 0)
            g_wait(1)
            compute_and_scatter(1)

            @pl.when(j < _NBATCH - 4)
            def _():
                idx_start(j + 3, 1)

        idx_wait(0)
        s_wait(0)
        g_start(0)
        g_wait(0)
        compute_and_scatter(0)
        s_wait(0)
        s_wait(1)

        plsc.subcore_barrier()

        for k in range(_STRIPE // _WCHUNK):
            r0 = sid * _STRIPE + k * _WCHUNK

            @pl.when(r0 < N)
            def _():
                pltpu.sync_copy(acc.at[pl.ds(r0, _WCHUNK)],
                                zbuf.at[pl.ds(0, _WCHUNK)])
                pltpu.sync_copy(zbuf.at[pl.ds(0, _WCHUNK)],
                                out_hbm.at[cid, pl.ds(r0, _WCHUNK)])

    return body(hw, ei_flat, att_flat)


def _tc_add_body(p_ref, o_ref):
    o_ref[...] = p_ref[0] + p_ref[1]


def _tc_add(partials):
    return pl.pallas_call(
        _tc_add_body,
        grid=(_TC_GRID,),
        in_specs=[pl.BlockSpec((2, _NROWS, D), lambda i: (0, i, 0))],
        out_specs=pl.BlockSpec((_NROWS, D), lambda i: (i, 0)),
        out_shape=jax.ShapeDtypeStruct((N, D), jnp.float32),
    )(partials)


def kernel(feat, edge_index, attention, W, b):
    ei_flat = edge_index.astype(jnp.int32).reshape(2 * E)
    hw = _tc_fwd(feat, W, b.reshape(1, D))
    partials = _sc_scatter(hw, ei_flat, attention.reshape(E))
    return _tc_add(partials)


# speedup vs baseline: 10.1798x; 1.1026x over previous
import dataclasses
import functools

import jax
import jax.numpy as jnp
from jax import lax
from jax.experimental import pallas as pl
from jax.experimental.pallas import tpu as pltpu
from jax.experimental.pallas import tpu_sc as plsc

N = 10000
E = 320000
D = 128


_TC_GRID = 25
_NROWS = N // _TC_GRID
_EROWS = E // _TC_GRID


def _tc_fwd_body(feat_ref, w_ref, b_ref, hw_ref):
    hw_ref[...] = (
        jnp.dot(feat_ref[...], w_ref[...], preferred_element_type=jnp.float32)
        + b_ref[...]
    )


def _tc_fwd(feat, w, b2d):
    return pl.pallas_call(
        _tc_fwd_body,
        grid=(_TC_GRID,),
        in_specs=[
            pl.BlockSpec((_NROWS, D), lambda i: (i, 0)),
            pl.BlockSpec((D, D), lambda i: (0, 0)),
            pl.BlockSpec((1, D), lambda i: (0, 0)),
        ],
        out_specs=pl.BlockSpec((_NROWS, D), lambda i: (i, 0)),
        out_shape=jax.ShapeDtypeStruct((N, D), jnp.float32),
    )(feat, w, b2d)


_GATHER_DNUMS = lax.GatherDimensionNumbers(
    offset_dims=(), collapsed_slice_dims=(0,), start_index_map=(0,))

_NTILES = 32
_EDGES_PER_TILE = E // _NTILES
_B = 80
_NBATCH = _EDGES_PER_TILE // _B
_ACC_ROWS = 10240
_STRIPE = _ACC_ROWS // 16
_ZROWS = 80
_WCHUNK = 80


def _sc_scatter(hw, ei_flat, att_flat):
    mesh = plsc.VectorSubcoreMesh(core_axis_name="c", subcore_axis_name="s")
    cp = pltpu.CompilerParams()
    if "needs_layout_passes" in pltpu.CompilerParams.__dataclass_fields__:
        cp = dataclasses.replace(cp, needs_layout_passes=False)

    @functools.partial(
        pl.kernel,
        out_type=jax.ShapeDtypeStruct((2, N, D), jnp.float32),
        mesh=mesh,
        compiler_params=cp,
        scratch_types=(
            [pltpu.VMEM((_B,), jnp.int32) for _ in range(4)]
            + [pltpu.VMEM((_B,), jnp.int32) for _ in range(4)]
            + [pltpu.VMEM((_B,), jnp.int32) for _ in range(4)]
            + [pltpu.VMEM((_B,), jnp.float32) for _ in range(4)]
            + [pltpu.VMEM((_B, D), jnp.float32) for _ in range(4)]
            + [pltpu.VMEM_SHARED((_ACC_ROWS, D), jnp.float32)]
            + [pltpu.SemaphoreType.DMA for _ in range(12)]
        ),
    )
    def body(hw_hbm, ei_hbm, att_hbm, out_hbm,
             sv0, sv1, sv2, sv3, dv0, dv1, dv2, dv3,
             ds0, ds1, ds2, ds3, av0, av1, av2, av3,
             rw0, rw1, rw2, rw3, acc,
             si0, si1, si2, si3, sg0, sg1, sg2, sg3, ss0, ss1, ss2, ss3):
        cid = lax.axis_index("c")
        sid = lax.axis_index("s")
        wid = cid * 16 + sid

        slots = tuple(zip((sv0, sv1, sv2, sv3), (dv0, dv1, dv2, dv3),
                          (ds0, ds1, ds2, ds3), (av0, av1, av2, av3),
                          (rw0, rw1, rw2, rw3), (si0, si1, si2, si3),
                          (sg0, sg1, sg2, sg3), (ss0, ss1, ss2, ss3)))

        @pl.loop(0, _B)
        def _(r):
            for c in range(D // 16):
                rw0[r, pl.ds(c * 16, 16)] = jnp.zeros((16,), jnp.float32)

        for k in range(_STRIPE // _B):
            r0 = sid * _STRIPE + k * _B
            pltpu.sync_copy(rw0, acc.at[pl.ds(r0, _B)])
        plsc.subcore_barrier()

        base = wid * _EDGES_PER_TILE

        def idx_start(j, s):
            srcv, dstv, _, attv, _, sem_i, _, _ = slots[s]
            off = base + j * _B
            pltpu.async_copy(ei_hbm.at[pl.ds(off, _B)], srcv, sem_i)
            pltpu.async_copy(ei_hbm.at[pl.ds(E + off, _B)], dstv, sem_i)
            pltpu.async_copy(att_hbm.at[pl.ds(off, _B)], attv, sem_i)

        def idx_wait(s):
            srcv, dstv, _, attv, _, sem_i, _, _ = slots[s]
            pltpu.make_async_copy(ei_hbm.at[pl.ds(0, _B)], srcv, sem_i).wait()
            pltpu.make_async_copy(ei_hbm.at[pl.ds(0, _B)], dstv, sem_i).wait()
            pltpu.make_async_copy(att_hbm.at[pl.ds(0, _B)], attv, sem_i).wait()

        def g_start(s):
            srcv, _, _, _, rows, _, sem_g, _ = slots[s]
            pltpu.async_copy(hw_hbm.at[srcv], rows, sem_g)

        def g_wait(s):
            srcv, _, _, _, rows, _, sem_g, _ = slots[s]
            pltpu.make_async_copy(hw_hbm.at[srcv], rows, sem_g).wait()

        def compute_and_scatter(s):
            _, dstv, dsts, attv, rows, _, _, sem_s = slots[s]

            @pl.loop(0, _B // 16)
            def _(g):
                a16 = attv[pl.ds(g * 16, 16)]
                for e2 in range(16):
                    a = lax.gather(
                        a16, jnp.full((16, 1), e2, jnp.int32),
                        _GATHER_DNUMS, slice_sizes=(1,),
                        mode=lax.GatherScatterMode.PROMISE_IN_BOUNDS)
                    e = g * 16 + e2
                    for c in range(D // 16):
                        rows[e, pl.ds(c * 16, 16)] = (
                            rows[e, pl.ds(c * 16, 16)] * a)

            for k in range(_B // 16):
                dsts[pl.ds(k * 16, 16)] = dstv[pl.ds(k * 16, 16)]
            pltpu.async_copy(rows, acc.at[dsts], sem_s, add=True)

        def s_wait(s):
            _, _, dsts, _, rows, _, _, sem_s = slots[s]
            pltpu.make_async_copy(rows, acc.at[dsts], sem_s).wait()

        for s in range(4):
            idx_start(s, s)
        for s in range(4):
            idx_wait(s)
            g_start(s)
        for s in range(4):
            g_wait(s)
            compute_and_scatter(s)
            idx_start(4 + s, s)

        @pl.loop(4, _NBATCH - 1, step=4)
        def _(j):
            for s in range(4):
                idx_wait(s)
                s_wait(s)
                g_start(s)
            for s in range(4):
                g_wait(s)
                compute_and_scatter(s)
                if s == 0:
                    idx_start(j + 4, 0)
                else:
                    @pl.when(j + 4 + s <= _NBATCH - 1)
                    def _():
                        idx_start(j + 4 + s, s)

        idx_wait(0)
        s_wait(0)
        g_start(0)
        g_wait(0)
        compute_and_scatter(0)
        for s in range(1, 4):
            s_wait(s)
        s_wait(0)

        plsc.subcore_barrier()

        for k in range(_STRIPE // _WCHUNK):
            r0 = sid * _STRIPE + k * _WCHUNK

            @pl.when(r0 < N)
            def _():
                pltpu.sync_copy(acc.at[pl.ds(r0, _WCHUNK)], rw0)
                pltpu.sync_copy(rw0, out_hbm.at[cid, pl.ds(r0, _WCHUNK)])

    return body(hw, ei_flat, att_flat)


def _tc_add_body(p_ref, o_ref):
    o_ref[...] = p_ref[0] + p_ref[1]


def _tc_add(partials):
    return pl.pallas_call(
        _tc_add_body,
        grid=(_TC_GRID,),
        in_specs=[pl.BlockSpec((2, _NROWS, D), lambda i: (0, i, 0))],
        out_specs=pl.BlockSpec((_NROWS, D), lambda i: (i, 0)),
        out_shape=jax.ShapeDtypeStruct((N, D), jnp.float32),
    )(partials)


def kernel(feat, edge_index, attention, W, b):
    ei_flat = edge_index.astype(jnp.int32).reshape(2 * E)
    hw = _tc_fwd(feat, W, b.reshape(1, D))
    partials = _sc_scatter(hw, ei_flat, attention.reshape(E))
    return _tc_add(partials)


# speedup vs baseline: 11.0834x; 1.0888x over previous
import dataclasses
import functools

import jax
import jax.numpy as jnp
from jax import lax
from jax.experimental import pallas as pl
from jax.experimental.pallas import tpu as pltpu
from jax.experimental.pallas import tpu_sc as plsc

N = 10000
E = 320000
D = 128


_TC_GRID = 10
_NROWS = N // _TC_GRID
_ADD_GRID = 5
_AROWS = N // _ADD_GRID


def _tc_fwd_body(feat_ref, w_ref, b_ref, hw_ref):
    hw_ref[...] = (
        jnp.dot(feat_ref[...], w_ref[...], preferred_element_type=jnp.float32)
        + b_ref[...]
    )


def _tc_fwd(feat, w, b2d):
    return pl.pallas_call(
        _tc_fwd_body,
        grid=(_TC_GRID,),
        in_specs=[
            pl.BlockSpec((_NROWS, D), lambda i: (i, 0)),
            pl.BlockSpec((D, D), lambda i: (0, 0)),
            pl.BlockSpec((1, D), lambda i: (0, 0)),
        ],
        out_specs=pl.BlockSpec((_NROWS, D), lambda i: (i, 0)),
        out_shape=jax.ShapeDtypeStruct((N, D), jnp.float32),
    )(feat, w, b2d)


_GATHER_DNUMS = lax.GatherDimensionNumbers(
    offset_dims=(), collapsed_slice_dims=(0,), start_index_map=(0,))

_NTILES = 32
_EDGES_PER_TILE = E // _NTILES
_B = 80
_NBATCH = _EDGES_PER_TILE // _B
_ACC_ROWS = 10240
_STRIPE = _ACC_ROWS // 16
_ZROWS = 80
_WCHUNK = 80


def _sc_scatter(hw, ei_flat, att_flat):
    mesh = plsc.VectorSubcoreMesh(core_axis_name="c", subcore_axis_name="s")
    cp = pltpu.CompilerParams()
    if "needs_layout_passes" in pltpu.CompilerParams.__dataclass_fields__:
        cp = dataclasses.replace(cp, needs_layout_passes=False)

    @functools.partial(
        pl.kernel,
        out_type=jax.ShapeDtypeStruct((2, N, D), jnp.float32),
        mesh=mesh,
        compiler_params=cp,
        scratch_types=(
            [pltpu.VMEM((_B,), jnp.int32) for _ in range(4)]
            + [pltpu.VMEM((_B,), jnp.int32) for _ in range(4)]
            + [pltpu.VMEM((_B,), jnp.int32) for _ in range(4)]
            + [pltpu.VMEM((_B,), jnp.float32) for _ in range(4)]
            + [pltpu.VMEM((_B, D), jnp.float32) for _ in range(4)]
            + [pltpu.VMEM_SHARED((_ACC_ROWS, D), jnp.float32)]
            + [pltpu.SemaphoreType.DMA for _ in range(12)]
        ),
    )
    def body(hw_hbm, ei_hbm, att_hbm, out_hbm,
             sv0, sv1, sv2, sv3, dv0, dv1, dv2, dv3,
             ds0, ds1, ds2, ds3, av0, av1, av2, av3,
             rw0, rw1, rw2, rw3, acc,
             si0, si1, si2, si3, sg0, sg1, sg2, sg3, ss0, ss1, ss2, ss3):
        cid = lax.axis_index("c")
        sid = lax.axis_index("s")
        wid = cid * 16 + sid

        slots = tuple(zip((sv0, sv1, sv2, sv3), (dv0, dv1, dv2, dv3),
                          (ds0, ds1, ds2, ds3), (av0, av1, av2, av3),
                          (rw0, rw1, rw2, rw3), (si0, si1, si2, si3),
                          (sg0, sg1, sg2, sg3), (ss0, ss1, ss2, ss3)))

        @pl.loop(0, _B)
        def _(r):
            for c in range(D // 16):
                rw0[r, pl.ds(c * 16, 16)] = jnp.zeros((16,), jnp.float32)

        for k in range(_STRIPE // _B):
            r0 = sid * _STRIPE + k * _B
            pltpu.sync_copy(rw0, acc.at[pl.ds(r0, _B)])
        plsc.subcore_barrier()

        base = wid * _EDGES_PER_TILE

        def idx_start(j, s):
            srcv, dstv, _, attv, _, sem_i, _, _ = slots[s]
            off = base + j * _B
            pltpu.async_copy(ei_hbm.at[pl.ds(off, _B)], srcv, sem_i)
            pltpu.async_copy(ei_hbm.at[pl.ds(E + off, _B)], dstv, sem_i)
            pltpu.async_copy(att_hbm.at[pl.ds(off, _B)], attv, sem_i)

        def idx_wait(s):
            srcv, dstv, _, attv, _, sem_i, _, _ = slots[s]
            pltpu.make_async_copy(ei_hbm.at[pl.ds(0, _B)], srcv, sem_i).wait()
            pltpu.make_async_copy(ei_hbm.at[pl.ds(0, _B)], dstv, sem_i).wait()
            pltpu.make_async_copy(att_hbm.at[pl.ds(0, _B)], attv, sem_i).wait()

        def g_start(s):
            srcv, _, _, _, rows, _, sem_g, _ = slots[s]
            pltpu.async_copy(hw_hbm.at[srcv], rows, sem_g)

        def g_wait(s):
            srcv, _, _, _, rows, _, sem_g, _ = slots[s]
            pltpu.make_async_copy(hw_hbm.at[srcv], rows, sem_g).wait()

        def compute_and_scatter(s):
            _, dstv, dsts, attv, rows, _, _, sem_s = slots[s]

            @pl.loop(0, _B // 16)
            def _(g):
                a16 = attv[pl.ds(g * 16, 16)]
                for e2 in range(16):
                    a = lax.gather(
                        a16, jnp.full((16, 1), e2, jnp.int32),
                        _GATHER_DNUMS, slice_sizes=(1,),
                        mode=lax.GatherScatterMode.PROMISE_IN_BOUNDS)
                    e = g * 16 + e2
                    for c in range(D // 16):
                        rows[e, pl.ds(c * 16, 16)] = (
                            rows[e, pl.ds(c * 16, 16)] * a)

            for k in range(_B // 16):
                dsts[pl.ds(k * 16, 16)] = dstv[pl.ds(k * 16, 16)]
            pltpu.async_copy(rows, acc.at[dsts], sem_s, add=True)

        def s_wait(s):
            _, _, dsts, _, rows, _, _, sem_s = slots[s]
            pltpu.make_async_copy(rows, acc.at[dsts], sem_s).wait()

        for s in range(4):
            idx_start(s, s)
        for s in range(4):
            idx_wait(s)
            g_start(s)
        for s in range(4):
            g_wait(s)
            compute_and_scatter(s)
            idx_start(4 + s, s)

        @pl.loop(4, _NBATCH - 1, step=4)
        def _(j):
            for s in range(4):
                idx_wait(s)
                s_wait(s)
                g_start(s)
            for s in range(4):
                g_wait(s)
                compute_and_scatter(s)
                if s == 0:
                    idx_start(j + 4, 0)
                else:
                    @pl.when(j + 4 + s <= _NBATCH - 1)
                    def _():
                        idx_start(j + 4 + s, s)

        idx_wait(0)
        s_wait(0)
        g_start(0)
        g_wait(0)
        compute_and_scatter(0)
        for s in range(1, 4):
            s_wait(s)
        s_wait(0)

        plsc.subcore_barrier()

        for k in range(_STRIPE // _WCHUNK):
            r0 = sid * _STRIPE + k * _WCHUNK

            @pl.when(r0 < N)
            def _():
                pltpu.sync_copy(acc.at[pl.ds(r0, _WCHUNK)], rw0)
                pltpu.sync_copy(rw0, out_hbm.at[cid, pl.ds(r0, _WCHUNK)])

    return body(hw, ei_flat, att_flat)


def _tc_add_body(p_ref, o_ref):
    o_ref[...] = p_ref[0] + p_ref[1]


def _tc_add(partials):
    return pl.pallas_call(
        _tc_add_body,
        grid=(_ADD_GRID,),
        in_specs=[pl.BlockSpec((2, _AROWS, D), lambda i: (0, i, 0))],
        out_specs=pl.BlockSpec((_AROWS, D), lambda i: (i, 0)),
        out_shape=jax.ShapeDtypeStruct((N, D), jnp.float32),
    )(partials)


def kernel(feat, edge_index, attention, W, b):
    ei_flat = edge_index.astype(jnp.int32).reshape(2 * E)
    hw = _tc_fwd(feat, W, b.reshape(1, D))
    partials = _sc_scatter(hw, ei_flat, attention.reshape(E))
    return _tc_add(partials)
